# baseline (speedup 1.0000x reference)
import jax
import jax.numpy as jnp
from jax.experimental import pallas as pl


def _pass1(av, s0c, s0, n, h, bi):
    two_n = av.shape[1]

    def body(a_ref, s0c_ref, s0_ref, row_ref, g_ref):
        k = pl.program_id(0)
        a = a_ref[...]
        row_ref[...] = jnp.dot(a, s0c_ref[...], preferred_element_type=jnp.float32)
        gt = jax.lax.dot_general(
            a, s0_ref[...],
            dimension_numbers=(((0,), (0,)), ((), ())),
            preferred_element_type=jnp.float32)

        @pl.when(k == 0)
        def _():
            g_ref[...] = gt

        @pl.when(k != 0)
        def _():
            g_ref[...] = g_ref[...] + gt

    return pl.pallas_call(
        body,
        grid=(n // bi,),
        in_specs=[
            pl.BlockSpec((bi, two_n), lambda k: (k, 0)),
            pl.BlockSpec((two_n, h), lambda k: (0, 0)),
            pl.BlockSpec((bi, h), lambda k: (k, 0)),
        ],
        out_specs=[
            pl.BlockSpec((bi, h), lambda k: (k, 0)),
            pl.BlockSpec((two_n, h), lambda k: (0, 0)),
        ],
        out_shape=[
            jax.ShapeDtypeStruct((n, h), jnp.float32),
            jax.ShapeDtypeStruct((two_n, h), jnp.float32),
        ],
    )(av, s0c, s0)


def _pass2(av, s1c, s1, bs, n, h, bk):
    def body(at_ref, aca_ref, acb_ref, s1c_ref, s1_ref, row_ref, ga_ref, gb_ref):
        k = pl.program_id(0)
        row_t = jnp.dot(at_ref[...], s1c_ref[...], preferred_element_type=jnp.float32)
        dn = (((0,), (0,)), ((), ()))
        ga_t = jax.lax.dot_general(aca_ref[...], s1_ref[...], dimension_numbers=dn,
                                   preferred_element_type=jnp.float32)
        gb_t = jax.lax.dot_general(acb_ref[...], s1_ref[...], dimension_numbers=dn,
                                   preferred_element_type=jnp.float32)

        @pl.when(k == 0)
        def _():
            row_ref[...] = row_t
            ga_ref[...] = ga_t
            gb_ref[...] = gb_t

        @pl.when(k != 0)
        def _():
            row_ref[...] = row_ref[...] + row_t
            ga_ref[...] = ga_ref[...] + ga_t
            gb_ref[...] = gb_ref[...] + gb_t

    nb = n // bs
    return pl.pallas_call(
        body,
        grid=(n // bk,),
        in_specs=[
            pl.BlockSpec((bs, 2 * bk), lambda k: (0, k)),
            pl.BlockSpec((bk, bs), lambda k: (k, 0)),
            pl.BlockSpec((bk, bs), lambda k: (k, nb)),
            pl.BlockSpec((2 * bk, h), lambda k: (k, 0)),
            pl.BlockSpec((bk, h), lambda k: (k, 0)),
        ],
        out_specs=[
            pl.BlockSpec((bs, h), lambda k: (0, 0)),
            pl.BlockSpec((bs, h), lambda k: (0, 0)),
            pl.BlockSpec((bs, h), lambda k: (0, 0)),
        ],
        out_shape=[
            jax.ShapeDtypeStruct((bs, h), jnp.float32),
            jax.ShapeDtypeStruct((bs, h), jnp.float32),
            jax.ShapeDtypeStruct((bs, h), jnp.float32),
        ],
    )(av, av, av, s1c, s1)


def _leaky(x):
    return jnp.where(x >= 0, x, 0.01 * x)


def _heads(ge, x_top, tweets, pe_w0, pe_b0, pe_wo, pe_bo,
           w1a, w1b, w1c, bc_b1, bc_w2, bc_b2):
    bs = tweets.shape[0]

    def body(ge_ref, xp_ref, tw_ref, pw0_ref, pb0_ref, pwo_ref, pbo_ref,
             w1a_ref, w1b_ref, w1c_ref, b1_ref, w2_ref, b2_ref, out_ref):
        hp = jnp.dot(xp_ref[...], pw0_ref[...], preferred_element_type=jnp.float32)
        hp = _leaky(hp + pb0_ref[...])
        prop = jnp.dot(hp, pwo_ref[...], preferred_element_type=jnp.float32) + pbo_ref[...]
        hid = (jnp.dot(ge_ref[...], w1a_ref[...], preferred_element_type=jnp.float32)
               + jnp.dot(prop, w1b_ref[...], preferred_element_type=jnp.float32)
               + jnp.dot(tw_ref[...], w1c_ref[...], preferred_element_type=jnp.float32)
               + b1_ref[...])
        hid = _leaky(hid)
        logits = _leaky(jnp.dot(hid, w2_ref[...], preferred_element_type=jnp.float32)
                        + b2_ref[...])
        m = jnp.max(logits, axis=-1, keepdims=True)
        e = jnp.exp(logits - m)
        out_ref[...] = e / jnp.sum(e, axis=-1, keepdims=True)

    return pl.pallas_call(
        body,
        out_shape=jax.ShapeDtypeStruct((bs, 2), jnp.float32),
    )(ge, x_top, tweets, pe_w0, pe_b0, pe_wo, pe_bo,
      w1a, w1b, w1c, bc_b1, bc_w2, bc_b2)


def kernel(x_feature, adj_matrix, des, tweets, batch_size,
           rel_w, gc_w0, gc_b0, gc_w1, gc_b1,
           pe_w0, pe_b0, pe_wo, pe_bo,
           bc_w1, bc_b1, bc_w2, bc_b2):
    n, f = x_feature.shape
    r = adj_matrix.shape[2]
    h = gc_w0.shape[1]
    bs, t = tweets.shape

    av = adj_matrix.transpose(0, 2, 1).reshape(n, r * n).astype(jnp.bfloat16)
    w = rel_w[:, 0]

    s0 = jnp.dot(x_feature, gc_w0)
    s0c = jnp.concatenate([w[0] * s0, w[1] * s0], axis=0).astype(jnp.bfloat16)
    row1, g1 = _pass1(av, s0c, s0.astype(jnp.bfloat16), n, h, bi=256)
    col1 = w[0] * g1[:n] + w[1] * g1[n:]
    out0 = row1 + col1 + gc_b0[None, :]

    s1 = jnp.dot(out0, gc_w1)
    s1c = jnp.concatenate([w[0] * s1, w[1] * s1], axis=0).astype(jnp.bfloat16)
    row2, g2a, g2b = _pass2(av, s1c, s1.astype(jnp.bfloat16), bs, n, h, bk=256)
    col2 = w[0] * g2a + w[1] * g2b
    out1_top = row2 + col2 + gc_b1[None, :]

    graph_emb = 0.5 * (out0[:bs] + out1_top)

    x_top = x_feature[:bs]
    return _heads(graph_emb, x_top, tweets,
                  pe_w0, pe_b0.reshape(1, h), pe_wo, pe_bo.reshape(1, h),
                  bc_w1[:h], bc_w1[h:2 * h], bc_w1[2 * h:],
                  bc_b1.reshape(1, h), bc_w2, bc_b2.reshape(1, 2))


# speedup vs baseline: 1.1753x; 1.1753x over previous
import jax
import jax.numpy as jnp
from jax.experimental import pallas as pl
from jax.experimental.pallas import tpu as pltpu


def _chunk_copy(v_hbm, buf, sems, slot, c, rows):
    return pltpu.make_async_copy(
        v_hbm.at[pl.ds(0, rows), c, :], buf.at[slot], sems.at[slot])


def _pass1(v3, s0cc, s0b, n, h):
    nc = v3.shape[1]

    def body(v_hbm, scc_ref, sb_ref, row_ref, g_ref, buf, sems):
        c = pl.program_id(0)

        @pl.when(c == 0)
        def _():
            _chunk_copy(v_hbm, buf, sems, 0, 0, n).start()

        @pl.when(c + 1 < nc)
        def _():
            _chunk_copy(v_hbm, buf, sems, (c + 1) % 2, c + 1, n).start()

        slot = c % 2
        _chunk_copy(v_hbm, buf, sems, slot, c, n).wait()
        a = buf[slot].astype(jnp.bfloat16)
        rt = jnp.dot(a, scc_ref[...], preferred_element_type=jnp.float32)
        g_ref[...] = jax.lax.dot_general(
            a, sb_ref[...], dimension_numbers=(((0,), (0,)), ((), ())),
            preferred_element_type=jnp.float32)

        @pl.when(c == 0)
        def _():
            row_ref[...] = rt

        @pl.when(c != 0)
        def _():
            row_ref[...] = row_ref[...] + rt

    return pl.pallas_call(
        body,
        grid=(nc,),
        in_specs=[
            pl.BlockSpec(memory_space=pltpu.MemorySpace.HBM),
            pl.BlockSpec((128, h), lambda c: (c, 0)),
            pl.BlockSpec((n, h), lambda c: (0, 0)),
        ],
        out_specs=[
            pl.BlockSpec((n, h), lambda c: (0, 0)),
            pl.BlockSpec((128, h), lambda c: (c, 0)),
        ],
        out_shape=[
            jax.ShapeDtypeStruct((n, h), jnp.float32),
            jax.ShapeDtypeStruct((nc * 128, h), jnp.float32),
        ],
        scratch_shapes=[
            pltpu.VMEM((2, n, 128), jnp.float32),
            pltpu.SemaphoreType.DMA((2,)),
        ],
    )(v3, s0cc, s0b)


def _pass2_row(v3, s1cc, bs, h):
    nc = v3.shape[1]

    def body(v_hbm, scc_ref, row_ref, buf, sems):
        c = pl.program_id(0)

        @pl.when(c == 0)
        def _():
            _chunk_copy(v_hbm, buf, sems, 0, 0, bs).start()

        @pl.when(c + 1 < nc)
        def _():
            _chunk_copy(v_hbm, buf, sems, (c + 1) % 2, c + 1, bs).start()

        slot = c % 2
        _chunk_copy(v_hbm, buf, sems, slot, c, bs).wait()
        a = buf[slot].astype(jnp.bfloat16)
        rt = jnp.dot(a, scc_ref[...], preferred_element_type=jnp.float32)

        @pl.when(c == 0)
        def _():
            row_ref[...] = rt

        @pl.when(c != 0)
        def _():
            row_ref[...] = row_ref[...] + rt

    return pl.pallas_call(
        body,
        grid=(nc,),
        in_specs=[
            pl.BlockSpec(memory_space=pltpu.MemorySpace.HBM),
            pl.BlockSpec((128, h), lambda c: (c, 0)),
        ],
        out_specs=pl.BlockSpec((bs, h), lambda c: (0, 0)),
        out_shape=jax.ShapeDtypeStruct((bs, h), jnp.float32),
        scratch_shapes=[
            pltpu.VMEM((2, bs, 128), jnp.float32),
            pltpu.SemaphoreType.DMA((2,)),
        ],
    )(v3, s1cc)


def _pass2_col(v3, s1b, n, h, ncc):
    def body(v_hbm, sb_ref, g_ref, buf, sems):
        c = pl.program_id(0)

        @pl.when(c == 0)
        def _():
            _chunk_copy(v_hbm, buf, sems, 0, 0, n).start()

        @pl.when(c + 1 < ncc)
        def _():
            _chunk_copy(v_hbm, buf, sems, (c + 1) % 2, c + 1, n).start()

        slot = c % 2
        _chunk_copy(v_hbm, buf, sems, slot, c, n).wait()
        a = buf[slot].astype(jnp.bfloat16)
        g_ref[...] = jax.lax.dot_general(
            a, sb_ref[...], dimension_numbers=(((0,), (0,)), ((), ())),
            preferred_element_type=jnp.float32)

    return pl.pallas_call(
        body,
        grid=(ncc,),
        in_specs=[
            pl.BlockSpec(memory_space=pltpu.MemorySpace.HBM),
            pl.BlockSpec((n, h), lambda c: (0, 0)),
        ],
        out_specs=pl.BlockSpec((128, h), lambda c: (c, 0)),
        out_shape=jax.ShapeDtypeStruct((ncc * 128, h), jnp.float32),
        scratch_shapes=[
            pltpu.VMEM((2, n, 128), jnp.float32),
            pltpu.SemaphoreType.DMA((2,)),
        ],
    )(v3, s1b)


def _leaky(x):
    return jnp.where(x >= 0, x, 0.01 * x)


def _heads(ge, x_top, tweets, pe_w0, pe_b0, pe_wo, pe_bo,
           w1a, w1b, w1c, bc_b1, bc_w2, bc_b2):
    bs = tweets.shape[0]

    def body(ge_ref, xp_ref, tw_ref, pw0_ref, pb0_ref, pwo_ref, pbo_ref,
             w1a_ref, w1b_ref, w1c_ref, b1_ref, w2_ref, b2_ref, out_ref):
        hp = jnp.dot(xp_ref[...], pw0_ref[...], preferred_element_type=jnp.float32)
        hp = _leaky(hp + pb0_ref[...])
        prop = jnp.dot(hp, pwo_ref[...], preferred_element_type=jnp.float32) + pbo_ref[...]
        hid = (jnp.dot(ge_ref[...], w1a_ref[...], preferred_element_type=jnp.float32)
               + jnp.dot(prop, w1b_ref[...], preferred_element_type=jnp.float32)
               + jnp.dot(tw_ref[...], w1c_ref[...], preferred_element_type=jnp.float32)
               + b1_ref[...])
        hid = _leaky(hid)
        logits = _leaky(jnp.dot(hid, w2_ref[...], preferred_element_type=jnp.float32)
                        + b2_ref[...])
        m = jnp.max(logits, axis=-1, keepdims=True)
        e = jnp.exp(logits - m)
        out_ref[...] = e / jnp.sum(e, axis=-1, keepdims=True)

    return pl.pallas_call(
        body,
        out_shape=jax.ShapeDtypeStruct((bs, 2), jnp.float32),
    )(ge, x_top, tweets, pe_w0, pe_b0, pe_wo, pe_bo,
      w1a, w1b, w1c, bc_b1, bc_w2, bc_b2)


def _chunk_weighted(s, w, n, h):
    r = w.shape[0]
    sr = s.reshape(n // 128, 1, 128, h) * w[None, :, None, None]
    return sr.reshape(n * r // 128 * 128, h).astype(jnp.bfloat16)


def _pair_combine(g, w, h):
    r = w.shape[0]
    return (g.reshape(-1, r, 128, h) * w[None, :, None, None]).sum(axis=1).reshape(-1, h)


def kernel(x_feature, adj_matrix, des, tweets, batch_size,
           rel_w, gc_w0, gc_b0, gc_w1, gc_b1,
           pe_w0, pe_b0, pe_wo, pe_bo,
           bc_w1, bc_b1, bc_w2, bc_b2):
    n, f = x_feature.shape
    r = adj_matrix.shape[2]
    h = gc_w0.shape[1]
    bs, t = tweets.shape

    v3 = adj_matrix.reshape(n, n // 128, 128, r).transpose(0, 1, 3, 2).reshape(n, n * r // 128, 128)
    w = rel_w[:, 0]

    s0 = jnp.dot(x_feature, gc_w0)
    s0cc = _chunk_weighted(s0, w, n, h)
    row1, g1 = _pass1(v3, s0cc, s0.astype(jnp.bfloat16), n, h)
    col1 = _pair_combine(g1, w, h)
    out0 = row1 + col1 + gc_b0[None, :]

    s1 = jnp.dot(out0, gc_w1)
    s1cc = _chunk_weighted(s1, w, n, h)
    row2 = _pass2_row(v3, s1cc, bs, h)
    g2 = _pass2_col(v3, s1.astype(jnp.bfloat16), n, h, ncc=bs * r // 128)
    col2 = _pair_combine(g2, w, h)
    out1_top = row2 + col2 + gc_b1[None, :]

    graph_emb = 0.5 * (out0[:bs] + out1_top)

    x_top = x_feature[:bs]
    return _heads(graph_emb, x_top, tweets,
                  pe_w0, pe_b0.reshape(1, h), pe_wo, pe_bo.reshape(1, h),
                  bc_w1[:h], bc_w1[h:2 * h], bc_w1[2 * h:],
                  bc_b1.reshape(1, h), bc_w2, bc_b2.reshape(1, 2))
